# Initial kernel scaffold; baseline (speedup 1.0000x reference)
import jax
import jax.numpy as jnp
from jax.experimental import pallas as pl


def kernel(x, y, perm, grad2):
    raise NotImplementedError("write your pallas kernel here")


# speedup vs baseline: 1225.3900x; 1225.3900x over previous
import functools

import jax
import jax.numpy as jnp
from jax import lax
from jax.experimental import pallas as pl
from jax.experimental.pallas import tpu as pltpu
from jax.experimental.pallas import tpu_sc as plsc

N = 4194304
NC = 2
NS = 16
L = 16
NW = NC * NS
PER_W = N // NW
CHUNK = 8192
NCHUNK = PER_W // CHUNK


def _fade_poly(t):
    return t * t * t * (t * (t * 6.0 - 15.0) + 10.0)


def _floor_parts(val):
    t = val.astype(jnp.int32)
    neg = val < t.astype(jnp.float32)
    i = jnp.where(neg, t - 1, t)
    f = val - i.astype(jnp.float32)
    return i, f


def _perlin_body(x_hbm, y_hbm, perm_hbm, g_hbm, out_hbm,
                 perm_v, g_v, gx_tab, gy_tab, xb, yb, ob):
    wid = lax.axis_index("s") * NC + lax.axis_index("c")

    pltpu.sync_copy(perm_hbm.at[pl.ds(0, 256)], perm_v)
    pltpu.sync_copy(g_hbm, g_v)

    for i in range(256 // L):
        pv = perm_v[pl.ds(i * L, L)]
        h2 = (pv & 7) * 2
        gx_tab[pl.ds(i * L, L)] = plsc.load_gather(g_v, [h2])
        gy_tab[pl.ds(i * L, L)] = plsc.load_gather(g_v, [h2 + 1])

    base_w = wid * PER_W

    def chunk_body(ci, carry):
        base = base_w + ci * CHUNK
        pltpu.sync_copy(x_hbm.at[pl.ds(base, CHUNK)], xb)
        pltpu.sync_copy(y_hbm.at[pl.ds(base, CHUNK)], yb)

        def vec_body(vi, c2):
            o = vi * L
            xv = xb[pl.ds(o, L)]
            yv = yb[pl.ds(o, L)]
            xi, xf = _floor_parts(xv)
            yi, yf = _floor_parts(yv)
            u = _fade_poly(xf)
            v = _fade_poly(yf)
            xi0 = xi & 255
            yi0 = yi & 255
            px0 = plsc.load_gather(perm_v, [xi0])
            px1 = plsc.load_gather(perm_v, [(xi0 + 1) & 255])
            iaa = (px0 + yi0) & 255
            iab = (px0 + yi0 + 1) & 255
            iba = (px1 + yi0) & 255
            ibb = (px1 + yi0 + 1) & 255
            xf1 = xf - 1.0
            yf1 = yf - 1.0
            g_aa = (plsc.load_gather(gx_tab, [iaa]) * xf
                    + plsc.load_gather(gy_tab, [iaa]) * yf)
            g_ab = (plsc.load_gather(gx_tab, [iab]) * xf
                    + plsc.load_gather(gy_tab, [iab]) * yf1)
            g_ba = (plsc.load_gather(gx_tab, [iba]) * xf1
                    + plsc.load_gather(gy_tab, [iba]) * yf)
            g_bb = (plsc.load_gather(gx_tab, [ibb]) * xf1
                    + plsc.load_gather(gy_tab, [ibb]) * yf1)
            r1 = g_aa + u * (g_ba - g_aa)
            r2 = g_ab + u * (g_bb - g_ab)
            ob[pl.ds(o, L)] = r1 + v * (r2 - r1)
            return c2

        lax.fori_loop(0, CHUNK // L, vec_body, 0)
        pltpu.sync_copy(ob, out_hbm.at[pl.ds(base, CHUNK)])
        return carry

    lax.fori_loop(0, NCHUNK, chunk_body, 0)


@jax.jit
def kernel(x, y, perm, grad2):
    grad_flat = grad2.reshape(16)
    mesh = plsc.VectorSubcoreMesh(core_axis_name="c", subcore_axis_name="s")
    fn = pl.kernel(
        _perlin_body,
        mesh=mesh,
        compiler_params=pltpu.CompilerParams(needs_layout_passes=False),
        out_type=jax.ShapeDtypeStruct((N,), jnp.float32),
        scratch_types=[
            pltpu.VMEM((256,), jnp.int32),
            pltpu.VMEM((16,), jnp.float32),
            pltpu.VMEM((256,), jnp.float32),
            pltpu.VMEM((256,), jnp.float32),
            pltpu.VMEM((CHUNK,), jnp.float32),
            pltpu.VMEM((CHUNK,), jnp.float32),
            pltpu.VMEM((CHUNK,), jnp.float32),
        ],
    )
    return fn(x, y, perm.astype(jnp.int32), grad_flat)


# speedup vs baseline: 2248.7984x; 1.8352x over previous
import jax
import jax.numpy as jnp
from jax import lax
from jax.experimental import pallas as pl
from jax.experimental.pallas import tpu as pltpu
from jax.experimental.pallas import tpu_sc as plsc

N = 4194304
NC = 2
NS = 16
L = 16
NW = NC * NS
PER_W = N // NW
CHUNK = 8192
NCHUNK = PER_W // CHUNK
NPAIR = NCHUNK // 2


def _fade_poly(t):
    return t * t * t * (t * (t * 6.0 - 15.0) + 10.0)


def _floor_parts(val):
    t = val.astype(jnp.int32)
    neg = val < t.astype(jnp.float32)
    i = jnp.where(neg, t - 1, t)
    f = val - i.astype(jnp.float32)
    return i, f


def _perlin_body(x_hbm, y_hbm, perm_hbm, g_hbm, out_hbm,
                 perm_v, g_v, gx_tab, gy_tab, xb, yb, ob,
                 in_sems, out_sems):
    wid = lax.axis_index("s") * NC + lax.axis_index("c")

    pltpu.sync_copy(perm_hbm, perm_v)
    pltpu.sync_copy(g_hbm, g_v)

    for i in range(512 // L):
        pv = perm_v[pl.ds(i * L, L)]
        h2 = (pv & 7) * 2
        gx_tab[pl.ds(i * L, L)] = plsc.load_gather(g_v, [h2])
        gy_tab[pl.ds(i * L, L)] = plsc.load_gather(g_v, [h2 + 1])

    base_w = wid * PER_W

    def start_in(ci, s):
        base = base_w + ci * CHUNK
        pltpu.async_copy(x_hbm.at[pl.ds(base, CHUNK)], xb.at[s], in_sems.at[s])
        pltpu.async_copy(y_hbm.at[pl.ds(base, CHUNK)], yb.at[s], in_sems.at[s])

    for s in range(2):
        start_in(s, s)

    def pair_body(pi, carry):
        for s in range(2):
            ci = pi * 2 + s
            base = base_w + ci * CHUNK
            pltpu.make_async_copy(
                x_hbm.at[pl.ds(base, CHUNK)], xb.at[s], in_sems.at[s]).wait()
            pltpu.make_async_copy(
                y_hbm.at[pl.ds(base, CHUNK)], yb.at[s], in_sems.at[s]).wait()
            @pl.when(pi > 0)
            def _():
                pltpu.make_async_copy(
                    ob.at[s], out_hbm.at[pl.ds(base, CHUNK)],
                    out_sems.at[s]).wait()

            @plsc.parallel_loop(0, CHUNK, step=L, unroll=4)
            def _(o):
                xv = xb[s, pl.ds(o, L)]
                yv = yb[s, pl.ds(o, L)]
                xi, xf = _floor_parts(xv)
                yi, yf = _floor_parts(yv)
                u = _fade_poly(xf)
                v = _fade_poly(yf)
                xi0 = xi & 255
                yi0 = yi & 255
                px0 = plsc.load_gather(perm_v, [xi0])
                px1 = plsc.load_gather(perm_v, [xi0 + 1])
                iaa = px0 + yi0
                iba = px1 + yi0
                xf1 = xf - 1.0
                yf1 = yf - 1.0
                g_aa = (plsc.load_gather(gx_tab, [iaa]) * xf
                        + plsc.load_gather(gy_tab, [iaa]) * yf)
                g_ab = (plsc.load_gather(gx_tab, [iaa + 1]) * xf
                        + plsc.load_gather(gy_tab, [iaa + 1]) * yf1)
                g_ba = (plsc.load_gather(gx_tab, [iba]) * xf1
                        + plsc.load_gather(gy_tab, [iba]) * yf)
                g_bb = (plsc.load_gather(gx_tab, [iba + 1]) * xf1
                        + plsc.load_gather(gy_tab, [iba + 1]) * yf1)
                r1 = g_aa + u * (g_ba - g_aa)
                r2 = g_ab + u * (g_bb - g_ab)
                ob[s, pl.ds(o, L)] = r1 + v * (r2 - r1)

            pltpu.async_copy(
                ob.at[s], out_hbm.at[pl.ds(base, CHUNK)], out_sems.at[s])

            @pl.when(pi < NPAIR - 1)
            def _():
                start_in(ci + 2, s)
        return carry

    lax.fori_loop(0, NPAIR, pair_body, 0)

    for s in range(2):
        base = base_w + (NCHUNK - 2 + s) * CHUNK
        pltpu.make_async_copy(
            ob.at[s], out_hbm.at[pl.ds(base, CHUNK)], out_sems.at[s]).wait()


@jax.jit
def kernel(x, y, perm, grad2):
    grad_flat = grad2.reshape(16)
    mesh = plsc.VectorSubcoreMesh(core_axis_name="c", subcore_axis_name="s")
    fn = pl.kernel(
        _perlin_body,
        mesh=mesh,
        compiler_params=pltpu.CompilerParams(needs_layout_passes=False),
        out_type=jax.ShapeDtypeStruct((N,), jnp.float32),
        scratch_types=[
            pltpu.VMEM((512,), jnp.int32),
            pltpu.VMEM((16,), jnp.float32),
            pltpu.VMEM((512,), jnp.float32),
            pltpu.VMEM((512,), jnp.float32),
            pltpu.VMEM((2, CHUNK), jnp.float32),
            pltpu.VMEM((2, CHUNK), jnp.float32),
            pltpu.VMEM((2, CHUNK), jnp.float32),
            pltpu.SemaphoreType.DMA((2,)),
            pltpu.SemaphoreType.DMA((2,)),
        ],
    )
    return fn(x, y, perm.astype(jnp.int32), grad_flat)
